# Initial kernel scaffold; baseline (speedup 1.0000x reference)
import jax
import jax.numpy as jnp
from jax.experimental import pallas as pl


def kernel(pred_logits, pred_boxes, pen_features, target_sizes):
    raise NotImplementedError("write your pallas kernel here")


# speedup vs baseline: 582.9341x; 582.9341x over previous
import functools

import jax
import jax.numpy as jnp
from jax import lax
from jax.experimental import pallas as pl
from jax.experimental.pallas import tpu as pltpu
from jax.experimental.pallas import tpu_sc as plsc

Q = 1000
C = 91
CP = 128
D = 256
K = Q * C
NW = 32
CHUNK = 2848
KPAD = NW * CHUNK
W = 128
NWIN = CHUNK // W + 1
NVREG = NWIN * W // 16
LROWS = 32


def _wid():
    return lax.axis_index("s") * 2 + lax.axis_index("c")


def _compact_body(prob_hbm, boxflat_hbm, ts_hbm, logflat_hbm,
                  compact_hbm, counts_hbm, btable_hbm, logpad_hbm,
                  probbuf, segbuf, cbuf, bbuf, btbuf, tsbuf, lbuf, lpbuf):
    wid = _wid()
    start = wid * CHUNK
    iota = lax.iota(jnp.int32, 16)

    pltpu.sync_copy(prob_hbm.at[pl.ds(start, CHUNK)], probbuf)

    def step(i, ptr):
        v = probbuf[pl.ds(i * 16, 16)]
        m = v > 0.1
        ids = start + i * 16 + iota
        mi = m.astype(jnp.int32)
        pos = ptr + plsc.cumsum(mi) - mi
        plsc.store_scatter(segbuf, [pos], ids, mask=m)
        return ptr + jnp.sum(mi)

    n = lax.fori_loop(0, CHUNK // 16, step, jnp.int32(0))

    pltpu.sync_copy(segbuf.at[pl.ds(0, CHUNK)],
                    compact_hbm.at[pl.ds(start, CHUNK)])
    cbuf[pl.ds(0, 16)] = jnp.zeros((16,), jnp.int32) + n
    pltpu.sync_copy(cbuf.at[pl.ds(0, 8)], counts_hbm.at[pl.ds(wid * 8, 8)])

    def pad_rows(nrows):
        pltpu.sync_copy(logflat_hbm.at[pl.ds(wid * (LROWS * C), nrows * C)],
                        lbuf.at[pl.ds(0, nrows * C)])

        def prow(k, _):
            for t in range(6):
                lpbuf[k, pl.ds(t * 16, 16)] = plsc.load_gather(
                    lbuf, [k * C + t * 16 + iota])
            return 0

        lax.fori_loop(0, nrows, prow, 0)
        pltpu.sync_copy(lpbuf.at[pl.ds(0, nrows)],
                        logpad_hbm.at[pl.ds(wid * LROWS, nrows)])

    @pl.when(wid < NW - 1)
    def _pad_full():
        pad_rows(LROWS)

    @pl.when(wid == NW - 1)
    def _pad_tail():
        pad_rows(Q - (NW - 1) * LROWS)

    @pl.when(wid == 0)
    def _box_table():
        pltpu.sync_copy(boxflat_hbm, bbuf)
        pltpu.sync_copy(ts_hbm, tsbuf)
        sidx = jnp.where(iota % 2 == 0, 1, 0)
        sv = plsc.load_gather(tsbuf, [sidx]).astype(jnp.float32)
        patt = (iota // 4) * 4 + (iota % 2)
        sgn = jnp.where((iota % 4) < 2, -0.5, 0.5).astype(jnp.float32)

        def bstep(i, _):
            base = i * 16
            a = plsc.load_gather(bbuf, [base + patt])
            d = plsc.load_gather(bbuf, [base + patt + 2])
            btbuf[pl.ds(base, 16)] = (a + sgn * d) * sv
            return 0

        lax.fori_loop(0, Q * 4 // 16, bstep, 0)
        pltpu.sync_copy(btbuf, btable_hbm)


def _gather_body(compact_hbm, counts_hbm, prob_hbm, bt_hbm, ob_hbm,
                 logpad_hbm, pen_hbm,
                 scores_hbm, labels_hbm, boxes_hbm, origb_hbm,
                 nlog_hbm, npen_hbm,
                 cntbuf, ebuf, srcbuf, idxbuf, ibuf, rbuf,
                 labbuf, scorebuf, btab, otab, boxbuf, obuf,
                 penwin, logwin, sem):
    wid = _wid()
    start = wid * CHUNK
    iota = lax.iota(jnp.int32, 16)

    pltpu.sync_copy(counts_hbm, cntbuf)
    c_lo = plsc.load_gather(cntbuf, [iota * 8])
    c_hi = plsc.load_gather(cntbuf, [128 + iota * 8])
    tot_lo = jnp.sum(c_lo)
    n_total = tot_lo + jnp.sum(c_hi)
    e_lo = plsc.cumsum(c_lo) - c_lo
    e_hi = plsc.cumsum(c_hi) - c_hi + tot_lo
    ebuf[pl.ds(0, 16)] = e_lo
    ebuf[pl.ds(16, 16)] = e_hi

    pltpu.sync_copy(bt_hbm, btab)
    pltpu.sync_copy(ob_hbm, otab)

    def src_step(i, _):
        p = start + i * 16 + iota
        s = jnp.zeros((16,), jnp.int32)
        for step in (16, 8, 4, 2, 1):
            t = s + step
            et = plsc.load_gather(ebuf, [t])
            s = jnp.where(et <= p, t, s)
        es = plsc.load_gather(ebuf, [s])
        src = s * CHUNK + (p - es)
        src = jnp.where(p < n_total, src, 0)
        srcbuf[i // 8, pl.ds((i % 8) * 16, 16)] = src
        return 0

    lax.fori_loop(0, NVREG, src_step, 0)

    def idx_step(j, _):
        pltpu.async_copy(compact_hbm.at[srcbuf.at[j]],
                         idxbuf.at[pl.ds(j * W, W)], sem).wait()
        return 0

    lax.fori_loop(0, NWIN, idx_step, 0)

    rcp = jnp.float32(1.0 / 91.0)

    def rl_step(i, _):
        p = start + i * 16 + iota
        iv = idxbuf[pl.ds(i * 16, 16)]
        iv = jnp.where(p < n_total, iv, 0)
        r0 = (iv.astype(jnp.float32) * rcp).astype(jnp.int32)
        lab0 = iv - r0 * C
        r = r0 + (lab0 >= C).astype(jnp.int32) - (lab0 < 0).astype(jnp.int32)
        lab = iv - r * C
        rbuf[i // 8, pl.ds((i % 8) * 16, 16)] = r
        ibuf[i // 8, pl.ds((i % 8) * 16, 16)] = iv
        labbuf[pl.ds(i * 16, 16)] = lab
        return 0

    lax.fori_loop(0, NVREG, rl_step, 0)

    def sc_step(j, _):
        pltpu.async_copy(prob_hbm.at[ibuf.at[j]],
                         scorebuf.at[pl.ds(j * W, W)], sem).wait()
        return 0

    lax.fori_loop(0, NWIN, sc_step, 0)

    def box_step(j, _):
        q = j * 16 + iota
        b = q >> 2
        comp = q & 3
        rb = plsc.load_gather(rbuf, [b >> 7, b & 127])
        addr = rb * 4 + comp
        boxbuf[pl.ds(j * 16, 16)] = plsc.load_gather(btab, [addr])
        obuf[pl.ds(j * 16, 16)] = plsc.load_gather(otab, [addr])
        return 0

    lax.fori_loop(0, CHUNK * 4 // 16, box_step, 0)

    def gather_window(j):
        cp1 = pltpu.async_copy(pen_hbm.at[rbuf.at[j]], penwin, sem)
        cp2 = pltpu.async_copy(logpad_hbm.at[rbuf.at[j]], logwin, sem)
        cp1.wait()
        cp2.wait()

    def write_window(j, rows):
        row0 = start + j * W
        pltpu.sync_copy(penwin.at[pl.ds(0, rows)],
                        npen_hbm.at[pl.ds(row0, rows)])
        pltpu.sync_copy(logwin.at[pl.ds(0, rows)],
                        nlog_hbm.at[pl.ds(row0, rows)])

    def do_range(nfull, tail, size):
        def wbody(j, _):
            gather_window(j)
            write_window(j, W)
            return 0

        lax.fori_loop(0, nfull, wbody, 0)
        gather_window(nfull)
        write_window(nfull, tail)
        pltpu.sync_copy(scorebuf.at[pl.ds(0, size)],
                        scores_hbm.at[pl.ds(start, size)])
        pltpu.sync_copy(labbuf.at[pl.ds(0, size)],
                        labels_hbm.at[pl.ds(start, size)])
        pltpu.sync_copy(boxbuf.at[pl.ds(0, size * 4)],
                        boxes_hbm.at[pl.ds(start * 4, size * 4)])
        pltpu.sync_copy(obuf.at[pl.ds(0, size * 4)],
                        origb_hbm.at[pl.ds(start * 4, size * 4)])

    @pl.when(wid < NW - 1)
    def _full():
        do_range(22, 32, CHUNK)

    @pl.when(wid == NW - 1)
    def _last():
        do_range(21, 24, K - (NW - 1) * CHUNK)


@functools.cache
def _build():
    mesh = plsc.VectorSubcoreMesh(core_axis_name="c", subcore_axis_name="s",
                                  num_cores=2, num_subcores=16)
    params = pltpu.CompilerParams(needs_layout_passes=False)

    compact_kernel = pl.kernel(
        _compact_body,
        out_type=(
            jax.ShapeDtypeStruct((KPAD,), jnp.int32),
            jax.ShapeDtypeStruct((NW * 8,), jnp.int32),
            jax.ShapeDtypeStruct((Q * 4,), jnp.float32),
            jax.ShapeDtypeStruct((Q, CP), jnp.float32),
        ),
        mesh=mesh,
        compiler_params=params,
        scratch_types=(
            pltpu.VMEM((CHUNK,), jnp.float32),
            pltpu.VMEM((CHUNK + 16,), jnp.int32),
            pltpu.VMEM((16,), jnp.int32),
            pltpu.VMEM((Q * 4,), jnp.float32),
            pltpu.VMEM((Q * 4,), jnp.float32),
            pltpu.VMEM((8,), jnp.int32),
            pltpu.VMEM((LROWS * C + 16,), jnp.float32),
            pltpu.VMEM((LROWS, CP), jnp.float32),
        ),
    )

    gather_kernel = pl.kernel(
        _gather_body,
        out_type=(
            jax.ShapeDtypeStruct((K,), jnp.float32),
            jax.ShapeDtypeStruct((K,), jnp.int32),
            jax.ShapeDtypeStruct((K * 4,), jnp.float32),
            jax.ShapeDtypeStruct((K * 4,), jnp.float32),
            jax.ShapeDtypeStruct((K, CP), jnp.float32),
            jax.ShapeDtypeStruct((K, D), jnp.float32),
        ),
        mesh=mesh,
        compiler_params=params,
        scratch_types=(
            pltpu.VMEM((NW * 8,), jnp.int32),
            pltpu.VMEM((32,), jnp.int32),
            pltpu.VMEM((NWIN, W), jnp.int32),
            pltpu.VMEM((NWIN * W,), jnp.int32),
            pltpu.VMEM((NWIN, W), jnp.int32),
            pltpu.VMEM((NWIN, W), jnp.int32),
            pltpu.VMEM((NWIN * W,), jnp.int32),
            pltpu.VMEM((NWIN * W,), jnp.float32),
            pltpu.VMEM((Q * 4,), jnp.float32),
            pltpu.VMEM((Q * 4,), jnp.float32),
            pltpu.VMEM((CHUNK * 4,), jnp.float32),
            pltpu.VMEM((CHUNK * 4,), jnp.float32),
            pltpu.VMEM((W, D), jnp.float32),
            pltpu.VMEM((W, CP), jnp.float32),
            pltpu.SemaphoreType.DMA,
        ),
    )
    return compact_kernel, gather_kernel


def kernel(pred_logits, pred_boxes, pen_features, target_sizes):
    compact_kernel, gather_kernel = _build()

    prob = jax.nn.sigmoid(pred_logits)
    flat = prob.reshape(-1)
    prob_pad = jnp.concatenate([flat, jnp.zeros((KPAD - K,), jnp.float32)])
    boxflat = pred_boxes.reshape(-1)
    logflat = pred_logits.reshape(-1)
    ts_pad = jnp.concatenate(
        [target_sizes.reshape(-1), jnp.zeros((6,), jnp.int32)])

    compact, counts, btflat, logpad = compact_kernel(
        prob_pad, boxflat, ts_pad, logflat)
    scores, labels, boxes, origb, nlog, npen = gather_kernel(
        compact, counts, prob_pad, btflat, boxflat,
        logpad, pen_features.reshape(Q, D))

    return (scores[None], labels[None], boxes.reshape(1, K, 4),
            origb.reshape(1, K, 4), nlog[:, :C][None], npen[None])
